# Initial kernel scaffold; baseline (speedup 1.0000x reference)
import jax
import jax.numpy as jnp
from jax.experimental import pallas as pl


def kernel(energy, W1, b1, W2, b2, W3, b3, batch):
    raise NotImplementedError("write your pallas kernel here")


# speedup vs baseline: 2.3054x; 2.3054x over previous
import jax
import jax.numpy as jnp
from jax.experimental import pallas as pl

_N = 50000
_D = 256
_M = 512
_B = 1000
_G = _N // _B


def _mlp_segsum_kernel(x_ref, w1_ref, b1_ref, w2_ref, b2_ref, w3_ref, b3_ref,
                       ids_ref, out_ref):
    i = pl.program_id(0)
    x = x_ref[...]
    h = jnp.dot(x, w1_ref[...], preferred_element_type=jnp.float32) + b1_ref[...]
    h = h * jax.nn.sigmoid(h)
    h = jnp.dot(h, w2_ref[...], preferred_element_type=jnp.float32) + b2_ref[...]
    h = h * jax.nn.sigmoid(h)
    s = jnp.sum(h * w3_ref[...], axis=1, keepdims=True) + b3_ref[0, 0]

    ids = ids_ref[0, 0, :]
    seg = jax.lax.broadcasted_iota(jnp.int32, (_B, _M), 1)
    hit = ids[:, None] == seg
    partial = jnp.sum(jnp.where(hit, s, 0.0), axis=0, keepdims=True)

    @pl.when(i == 0)
    def _():
        out_ref[...] = jnp.zeros_like(out_ref)

    out_ref[...] += partial


def kernel(energy, W1, b1, W2, b2, W3, b3, batch):
    ids3 = batch.astype(jnp.int32).reshape(_G, 1, _B)
    out = pl.pallas_call(
        _mlp_segsum_kernel,
        grid=(_G,),
        in_specs=[
            pl.BlockSpec((_B, _D), lambda i: (i, 0)),
            pl.BlockSpec((_D, _D), lambda i: (0, 0)),
            pl.BlockSpec((1, _D), lambda i: (0, 0)),
            pl.BlockSpec((_D, _D), lambda i: (0, 0)),
            pl.BlockSpec((1, _D), lambda i: (0, 0)),
            pl.BlockSpec((1, _D), lambda i: (0, 0)),
            pl.BlockSpec((1, 1), lambda i: (0, 0)),
            pl.BlockSpec((1, 1, _B), lambda i: (i, 0, 0)),
        ],
        out_specs=pl.BlockSpec((1, _M), lambda i: (0, 0)),
        out_shape=jax.ShapeDtypeStruct((1, _M), jnp.float32),
    )(energy, W1, b1.reshape(1, _D), W2, b2.reshape(1, _D),
      W3.reshape(1, _D), b3.reshape(1, 1), ids3)
    return out[0]
